# baseline (speedup 1.0000x reference)
import functools

import jax
import jax.numpy as jnp
from jax import lax
from jax.experimental import pallas as pl
from jax.experimental.pallas import tpu as pltpu
from jax.experimental.pallas import tpu_sc as plsc

N_NEG = 16
NC, NS = 2, 16
NW = NC * NS
NCHUNK = 4


def _sc_gather(table, idx, Bc, base_b):
    D = table.shape[1]
    Sp = idx.shape[1]
    b_per_w = Bc // NW
    npairs = b_per_w // 2
    mesh = plsc.VectorSubcoreMesh(core_axis_name="c", subcore_axis_name="s")

    @functools.partial(
        pl.kernel,
        mesh=mesh,
        compiler_params=pltpu.CompilerParams(use_tc_tiling_on_sc=False),
        out_type=jax.ShapeDtypeStruct((Bc, Sp, 2 * D), table.dtype),
        scratch_types=[
            pltpu.VMEM((b_per_w, Sp), jnp.int32),
            pltpu.VMEM((Sp, D), table.dtype),
            pltpu.VMEM((Sp, D), table.dtype),
            pltpu.SemaphoreType.DMA,
            pltpu.SemaphoreType.DMA,
        ],
    )
    def k(table_hbm, idx_hbm, out_hbm, idx_v, rows0, rows1, sem0, sem1):
        wid = lax.axis_index("s") * NC + lax.axis_index("c")
        b0 = wid * b_per_w
        pltpu.sync_copy(idx_hbm.at[pl.ds(base_b + b0, b_per_w)], idx_v)

        def gather(i, rows, sem):
            pltpu.async_copy(table_hbm.at[idx_v.at[i]], rows, sem)

        def store(i, rows):
            pltpu.sync_copy(
                rows, out_hbm.at[b0 + i, pl.ds(0, Sp), pl.ds(0, D)])

        gather(0, rows0, sem0)

        def pair(p, carry):
            i = 2 * p
            gather(i + 1, rows1, sem1)
            pltpu.make_async_copy(
                table_hbm.at[idx_v.at[i]], rows0, sem0).wait()
            store(i, rows0)

            @pl.when(p + 1 < npairs)
            def _():
                gather(i + 2, rows0, sem0)

            pltpu.make_async_copy(
                table_hbm.at[idx_v.at[i]], rows1, sem1).wait()
            store(i + 1, rows1)
            return carry

        lax.fori_loop(0, npairs, pair, 0)

    return k(table, idx)


def _tc_loss(E, x, n_samples, base_b, Bc, bblk=8):
    L, D = x.shape[1], x.shape[2]
    Sp = E.shape[1]

    def body(e_ref, x_ref, o_ref):
        xb = x_ref[...]
        xss = jnp.sum(xb * xb, axis=1, keepdims=True)
        xn = xb * jnp.minimum(lax.rsqrt(xss), 1e12)
        logits = []
        for k in range(n_samples):
            Ek = e_ref[:, k * L:(k + 1) * L, :D]
            ess = jnp.sum(Ek * Ek, axis=1, keepdims=True)
            rn = jnp.minimum(lax.rsqrt(ess), 1e12)
            logits.append(jnp.sum(Ek * rn * xn, axis=2))
        m = logits[0]
        for lk in logits[1:]:
            m = jnp.maximum(m, lk)
        s = jnp.exp(logits[0] - m)
        for lk in logits[1:]:
            s = s + jnp.exp(lk - m)
        o_ref[...] = m + jnp.log(s) - logits[0]

    boff = base_b // bblk
    return pl.pallas_call(
        body,
        grid=(Bc // bblk,),
        in_specs=[
            pl.BlockSpec((bblk, Sp, 2 * D), lambda b: (b, 0, 0)),
            pl.BlockSpec((bblk, L, D), lambda b: (b + boff, 0, 0)),
        ],
        out_specs=pl.BlockSpec((bblk, L), lambda b: (b, 0)),
        out_shape=jax.ShapeDtypeStruct((Bc, L), jnp.float32),
    )(E, x)


def kernel(input, target, W):
    B, L, D = input.shape
    V = W.shape[0]
    S = (1 + N_NEG) * L
    Sp = (S + 7) // 8 * 8
    neg_key = jax.random.key(42)
    neg = jax.random.randint(neg_key, (B, L, N_NEG), 0, V - 1, dtype=jnp.int32)
    negc = jnp.pad(jnp.transpose(neg, (0, 2, 1)).reshape(B, N_NEG * L),
                   ((0, 0), (L, Sp - S)))
    is_tgt = (jnp.arange(Sp) < L)[None, :]
    ttile = jnp.concatenate(
        [target] * (1 + N_NEG) + [jnp.zeros((B, Sp - S), jnp.int32)], axis=1)
    idx = jnp.where(is_tgt, ttile,
                    negc + (negc >= ttile).astype(jnp.int32))
    Bc = B // NCHUNK
    losses = []
    for c in range(NCHUNK):
        E_c = _sc_gather(W, idx, Bc, c * Bc)
        losses.append(_tc_loss(E_c, input, 1 + N_NEG, c * Bc, Bc))
    return jnp.concatenate(losses, axis=0)


# speedup vs baseline: 1.0313x; 1.0313x over previous
import functools

import jax
import jax.numpy as jnp
from jax import lax
from jax.experimental import pallas as pl
from jax.experimental.pallas import tpu as pltpu
from jax.experimental.pallas import tpu_sc as plsc

N_NEG = 16
NC, NS = 2, 16
NW = NC * NS
NCHUNK = 4


def _sc_gather(table, idx_flat, Bc, Sp, base_b):
    D = table.shape[1]
    b_per_w = Bc // NW
    npairs = b_per_w // 2
    mesh = plsc.VectorSubcoreMesh(core_axis_name="c", subcore_axis_name="s")

    @functools.partial(
        pl.kernel,
        mesh=mesh,
        compiler_params=pltpu.CompilerParams(use_tc_tiling_on_sc=False),
        out_type=jax.ShapeDtypeStruct((Bc, Sp, 2 * D), table.dtype),
        scratch_types=[
            pltpu.VMEM((b_per_w * Sp,), jnp.int32),
            pltpu.VMEM((Sp, D), table.dtype),
            pltpu.VMEM((Sp, D), table.dtype),
            pltpu.SemaphoreType.DMA,
            pltpu.SemaphoreType.DMA,
        ],
    )
    def k(table_hbm, idx_hbm, out_hbm, idx_v, rows0, rows1, sem0, sem1):
        wid = lax.axis_index("s") * NC + lax.axis_index("c")
        b0 = wid * b_per_w
        pltpu.sync_copy(
            idx_hbm.at[pl.ds((base_b + b0) * Sp, b_per_w * Sp)], idx_v)

        def gather(i, rows, sem):
            pltpu.async_copy(
                table_hbm.at[idx_v.at[pl.ds(i * Sp, Sp)]], rows, sem)

        def store(i, rows):
            pltpu.sync_copy(
                rows, out_hbm.at[b0 + i, pl.ds(0, Sp), pl.ds(0, D)])

        gather(0, rows0, sem0)

        def pair(p, carry):
            i = 2 * p
            gather(i + 1, rows1, sem1)
            pltpu.make_async_copy(
                table_hbm.at[idx_v.at[pl.ds(i * Sp, Sp)]], rows0, sem0).wait()
            store(i, rows0)

            @pl.when(p + 1 < npairs)
            def _():
                gather(i + 2, rows0, sem0)

            pltpu.make_async_copy(
                table_hbm.at[idx_v.at[pl.ds(i * Sp, Sp)]], rows1, sem1).wait()
            store(i + 1, rows1)
            return carry

        lax.fori_loop(0, npairs, pair, 0)

    return k(table, idx_flat)


def _tc_loss(E, x, n_samples, base_b, Bc, bblk=8):
    L, D = x.shape[1], x.shape[2]
    Sp = E.shape[1]

    def body(e_ref, x_ref, o_ref):
        xb = x_ref[...]
        xss = jnp.sum(xb * xb, axis=1, keepdims=True)
        xn = xb * jnp.minimum(lax.rsqrt(xss), 1e12)
        logits = []
        for k in range(n_samples):
            Ek = e_ref[:, k * L:(k + 1) * L, :D]
            ess = jnp.sum(Ek * Ek, axis=1, keepdims=True)
            rn = jnp.minimum(lax.rsqrt(ess), 1e12)
            logits.append(jnp.sum(Ek * rn * xn, axis=2))
        m = logits[0]
        for lk in logits[1:]:
            m = jnp.maximum(m, lk)
        s = jnp.exp(logits[0] - m)
        for lk in logits[1:]:
            s = s + jnp.exp(lk - m)
        o_ref[...] = m + jnp.log(s) - logits[0]

    boff = base_b // bblk
    return pl.pallas_call(
        body,
        grid=(Bc // bblk,),
        in_specs=[
            pl.BlockSpec((bblk, Sp, 2 * D), lambda b: (b, 0, 0)),
            pl.BlockSpec((bblk, L, D), lambda b: (b + boff, 0, 0)),
        ],
        out_specs=pl.BlockSpec((bblk, L), lambda b: (b, 0)),
        out_shape=jax.ShapeDtypeStruct((Bc, L), jnp.float32),
    )(E, x)


def kernel(input, target, W):
    B, L, D = input.shape
    V = W.shape[0]
    S = (1 + N_NEG) * L
    Sp = (S + 7) // 8 * 8
    neg_key = jax.random.key(42)
    neg = jax.random.randint(neg_key, (B, L, N_NEG), 0, V - 1, dtype=jnp.int32)
    neg_t = jnp.transpose(neg, (0, 2, 1))
    neg_t = neg_t + (neg_t >= target[:, None, :]).astype(jnp.int32)
    idx = jnp.concatenate([target[:, None, :], neg_t], axis=1)
    idx = idx.reshape(B, S)
    idx = jnp.pad(idx, ((0, 0), (0, Sp - S))).reshape(B * Sp)
    Bc = B // NCHUNK
    losses = []
    for c in range(NCHUNK):
        E_c = _sc_gather(W, idx, Bc, Sp, c * Bc)
        losses.append(_tc_loss(E_c, input, 1 + N_NEG, c * Bc, Bc))
    return jnp.concatenate(losses, axis=0)
